# baseline (speedup 1.0000x reference)
import functools

import jax
import jax.numpy as jnp
from jax import lax
from jax.experimental import pallas as pl
from jax.experimental.pallas import tpu as pltpu
from jax.experimental.pallas import tpu_sc as plsc

B, N, L, K = 4, 2048, 256, 16
ROWS = 256

NC, NS, LN = 2, 16, 16
NW = NC * NS
WPB = NW // B
RPW = N // WPB
GRP = RPW // LN


def _scores_body(x_ref, wa_ref, ba_ref, wb_ref, bb_ref, wc_ref, bc_ref,
                 out_ref, flat_ref):
    xb = x_ref[0]
    a = jnp.tanh(jnp.dot(xb, wa_ref[...], preferred_element_type=jnp.float32)
                 + ba_ref[...][None, :])
    b = jax.nn.sigmoid(jnp.dot(xb, wb_ref[...], preferred_element_type=jnp.float32)
                       + bb_ref[...][None, :])
    logits = (jnp.dot(a * b, wc_ref[...], preferred_element_type=jnp.float32)
              + bc_ref[...][None, :])
    m = jnp.max(logits, axis=0, keepdims=True)
    e = jnp.exp(logits - m)
    s = jnp.sum(e, axis=0, keepdims=True)
    sm = e / s
    out_ref[...] = sm[None]
    flat_ref[...] = jnp.reshape(sm, (N,))


def _ranks_body(acol_ref, arow_ref, rank_ref, rs_ref, re_ref):
    ac = acol_ref[0]
    aa = arow_ref[0]
    t = pl.program_id(1)
    jj = lax.broadcasted_iota(jnp.int32, (ROWS, N), 1)
    ii = lax.broadcasted_iota(jnp.int32, (ROWS, N), 0) + t * ROWS
    eq = aa == ac
    ones = jnp.ones((N, 1), jnp.float32)
    ltf = jnp.where(aa < ac, 1.0, 0.0)
    eqf = jnp.where(eq, 1.0, 0.0)
    eqltf = jnp.where(eq & (jj < ii), 1.0, 0.0)
    lt_sum = jnp.dot(ltf, ones, preferred_element_type=jnp.float32)
    eq_sum = jnp.dot(eqf, ones, preferred_element_type=jnp.float32)
    eqlt_sum = jnp.dot(eqltf, ones, preferred_element_type=jnp.float32)
    rank_ref[...] = jnp.reshape(lt_sum + eqlt_sum, (ROWS,)).astype(jnp.int32)
    rs_ref[...] = jnp.reshape(lt_sum, (ROWS,)).astype(jnp.int32)
    re_ref[...] = jnp.reshape(lt_sum + eq_sum - 1.0, (ROWS,)).astype(jnp.int32)


def _sc_knn_body(vals_hbm, ranks_hbm, rs_hbm, re_hbm, out_hbm,
                 vals_v, ranks_v, rs_src_v, re_src_v,
                 sval_v, sidx_v, srs_v, sre_v, otile_v, ctile_v):
    cid = lax.axis_index("c")
    sid = lax.axis_index("s")
    wid = sid * NC + cid
    b = lax.rem(wid, B)
    seg = lax.div(wid, B)

    pltpu.sync_copy(vals_hbm.at[pl.ds(b * N, N)], vals_v)
    pltpu.sync_copy(ranks_hbm.at[pl.ds(b * N, N)], ranks_v)
    pltpu.sync_copy(rs_hbm.at[pl.ds(b * N, N)], rs_src_v)
    pltpu.sync_copy(re_hbm.at[pl.ds(b * N, N)], re_src_v)

    def scat_body(i, carry):
        r = ranks_v[pl.ds(i * LN, LN)]
        idx = lax.iota(jnp.int32, LN) + i * LN
        plsc.store_scatter(sval_v, [r], vals_v[pl.ds(i * LN, LN)])
        plsc.store_scatter(sidx_v, [r], idx)
        plsc.store_scatter(srs_v, [r], rs_src_v[pl.ds(i * LN, LN)])
        plsc.store_scatter(sre_v, [r], re_src_v[pl.ds(i * LN, LN)])
        return carry

    lax.fori_loop(0, N // LN, scat_body, 0)

    lane = lax.iota(jnp.int32, LN)
    inf = jnp.full((LN,), jnp.inf, jnp.float32)

    def grp_body(g, carry):
        base = seg * RPW + g * LN
        vi = vals_v[pl.ds(base, LN)]
        p = ranks_v[pl.ds(base, LN)]
        l = p
        h = p + 1
        rows = base + lane
        for t in range(K):
            lvalid = l >= 0
            hvalid = h < N
            lc = jnp.maximum(l, 0)
            hc = jnp.minimum(h, N - 1)
            vl = plsc.load_gather(sval_v, [lc])
            vh = plsc.load_gather(sval_v, [hc])
            dl = jnp.where(lvalid, jnp.abs(vi - vl), inf)
            dh = jnp.where(hvalid, jnp.abs(vi - vh), inf)
            rs_l = plsc.load_gather(srs_v, [lc])
            re_l = plsc.load_gather(sre_v, [lc])
            eff = rs_l + (jnp.minimum(re_l, p) - lc)
            il = plsc.load_gather(sidx_v, [eff])
            ih = plsc.load_gather(sidx_v, [hc])
            pick_l = (dl < dh) | ((dl == dh) & (il < ih))
            picked = jnp.where(pick_l, il, ih)
            plsc.store_scatter(otile_v, [lane * K + t], picked)
            plsc.store_scatter(ctile_v, [lane * K + t], rows)
            l = jnp.where(pick_l, l - 1, l)
            h = jnp.where(pick_l, h, h + 1)
        pltpu.sync_copy(otile_v, out_hbm.at[pl.ds((b * N + base) * K, LN * K)])
        pltpu.sync_copy(ctile_v,
                        out_hbm.at[pl.ds((B * N + b * N + base) * K, LN * K)])
        return carry

    lax.fori_loop(0, GRP, grp_body, 0)


@jax.jit
def kernel(x, Wa, ba, Wb, bb, Wc, bc):
    scores, scores_flat = pl.pallas_call(
        _scores_body,
        grid=(B,),
        in_specs=[
            pl.BlockSpec((1, N, L), lambda b: (b, 0, 0)),
            pl.BlockSpec((L, L), lambda b: (0, 0)),
            pl.BlockSpec((L,), lambda b: (0,)),
            pl.BlockSpec((L, L), lambda b: (0, 0)),
            pl.BlockSpec((L,), lambda b: (0,)),
            pl.BlockSpec((L, 1), lambda b: (0, 0)),
            pl.BlockSpec((1,), lambda b: (0,)),
        ],
        out_specs=[
            pl.BlockSpec((1, N, 1), lambda b: (b, 0, 0)),
            pl.BlockSpec((N,), lambda b: (b,)),
        ],
        out_shape=[
            jax.ShapeDtypeStruct((B, N, 1), jnp.float32),
            jax.ShapeDtypeStruct((B * N,), jnp.float32),
        ],
    )(x, Wa, ba, Wb, bb, Wc, bc)

    a_col = scores
    a_row = scores.reshape(B, 1, N)

    ranks, run_start, run_end = pl.pallas_call(
        _ranks_body,
        grid=(B, N // ROWS),
        in_specs=[
            pl.BlockSpec((1, ROWS, 1), lambda b, t: (b, t, 0)),
            pl.BlockSpec((1, 1, N), lambda b, t: (b, 0, 0)),
        ],
        out_specs=[
            pl.BlockSpec((ROWS,), lambda b, t: (b * (N // ROWS) + t,)),
            pl.BlockSpec((ROWS,), lambda b, t: (b * (N // ROWS) + t,)),
            pl.BlockSpec((ROWS,), lambda b, t: (b * (N // ROWS) + t,)),
        ],
        out_shape=[
            jax.ShapeDtypeStruct((B * N,), jnp.int32),
            jax.ShapeDtypeStruct((B * N,), jnp.int32),
            jax.ShapeDtypeStruct((B * N,), jnp.int32),
        ],
    )(a_col, a_row)

    sc_knn = pl.kernel(
        _sc_knn_body,
        out_type=jax.ShapeDtypeStruct((2 * B * N * K,), jnp.int32),
        mesh=plsc.VectorSubcoreMesh(core_axis_name="c", subcore_axis_name="s"),
        compiler_params=pltpu.CompilerParams(needs_layout_passes=False),
        scratch_types=[
            pltpu.VMEM((N,), jnp.float32),
            pltpu.VMEM((N,), jnp.int32),
            pltpu.VMEM((N,), jnp.int32),
            pltpu.VMEM((N,), jnp.int32),
            pltpu.VMEM((N,), jnp.float32),
            pltpu.VMEM((N,), jnp.int32),
            pltpu.VMEM((N,), jnp.int32),
            pltpu.VMEM((N,), jnp.int32),
            pltpu.VMEM((LN * K,), jnp.int32),
            pltpu.VMEM((LN * K,), jnp.int32),
        ],
    )
    out_flat = sc_knn(scores_flat, ranks, run_start, run_end)

    return out_flat.reshape(2, B, N, K)


# speedup vs baseline: 1.1936x; 1.1936x over previous
import jax
import jax.numpy as jnp
from jax import lax
from jax.experimental import pallas as pl
from jax.experimental.pallas import tpu as pltpu
from jax.experimental.pallas import tpu_sc as plsc

B, N, L, K = 4, 2048, 256, 16
ROWS = 256

NC, NS, LN = 2, 16, 16
NW = NC * NS
WPB = NW // B
RPW = N // WPB
GRP = RPW // LN


def _scores_body(x_ref, wa_ref, ba_ref, wb_ref, bb_ref, wc_ref, bc_ref, out_ref):
    xb = x_ref[0]
    a = jnp.tanh(jnp.dot(xb, wa_ref[...], preferred_element_type=jnp.float32)
                 + ba_ref[...][None, :])
    b = jax.nn.sigmoid(jnp.dot(xb, wb_ref[...], preferred_element_type=jnp.float32)
                       + bb_ref[...][None, :])
    logits = (jnp.dot(a * b, wc_ref[...], preferred_element_type=jnp.float32)
              + bc_ref[...][None, :])
    m = jnp.max(logits, axis=0, keepdims=True)
    e = jnp.exp(logits - m)
    s = jnp.sum(e, axis=0, keepdims=True)
    out_ref[...] = (e / s)[None]


def _ranks_body(acol_ref, arow_ref, rank_ref, rse_ref):
    ac = acol_ref[0]
    aa = arow_ref[0]
    t = pl.program_id(1)
    jj = lax.broadcasted_iota(jnp.int32, (ROWS, N), 1)
    ii = lax.broadcasted_iota(jnp.int32, (ROWS, N), 0) + t * ROWS
    eq = aa == ac
    lt = (aa < ac).astype(jnp.int32)
    eq_lt = (eq & (jj < ii)).astype(jnp.int32)
    lt_sum = jnp.sum(lt, axis=1)
    eq_sum = jnp.sum(eq.astype(jnp.int32), axis=1)
    eqlt_sum = jnp.sum(eq_lt, axis=1)
    rank_ref[0] = (lt_sum + eqlt_sum)[:, None]
    rse_ref[0] = (lt_sum * 4096 + (lt_sum + eq_sum - 1))[:, None]


def _sc_knn_body(vals_hbm, ranks_hbm, rse_hbm, out_hbm,
                 vals_v, ranks_v, rse_src_v,
                 sval_v, sidx_v, srse_v, otile_v):
    cid = lax.axis_index("c")
    sid = lax.axis_index("s")
    wid = sid * NC + cid
    b = lax.rem(wid, B)
    seg = lax.div(wid, B)

    pltpu.sync_copy(vals_hbm.at[pl.ds(b * N, N)], vals_v)
    pltpu.sync_copy(ranks_hbm.at[pl.ds(b * N, N)], ranks_v)
    pltpu.sync_copy(rse_hbm.at[pl.ds(b * N, N)], rse_src_v)

    def scat_body(i, carry):
        r = ranks_v[pl.ds(i * LN, LN)]
        idx = lax.iota(jnp.int32, LN) + i * LN
        plsc.store_scatter(sval_v, [r], vals_v[pl.ds(i * LN, LN)])
        plsc.store_scatter(sidx_v, [r], idx)
        plsc.store_scatter(srse_v, [r], rse_src_v[pl.ds(i * LN, LN)])
        return carry

    lax.fori_loop(0, N // LN, scat_body, 0)

    lane = lax.iota(jnp.int32, LN)
    inf = jnp.full((LN,), jnp.inf, jnp.float32)

    def grp_body(g, carry):
        base = seg * RPW + g * LN
        vi = vals_v[pl.ds(base, LN)]
        p = ranks_v[pl.ds(base, LN)]
        l = p
        h = p + 1
        for t in range(K):
            lvalid = l >= 0
            hvalid = h < N
            lc = jnp.maximum(l, 0)
            hc = jnp.minimum(h, N - 1)
            vl = plsc.load_gather(sval_v, [lc])
            vh = plsc.load_gather(sval_v, [hc])
            dl = jnp.where(lvalid, jnp.abs(vi - vl), inf)
            dh = jnp.where(hvalid, jnp.abs(vi - vh), inf)
            rse_l = plsc.load_gather(srse_v, [lc])
            rs_l = jnp.right_shift(rse_l, 12)
            re_l = jnp.bitwise_and(rse_l, 4095)
            eff = rs_l + (jnp.minimum(re_l, p) - lc)
            il = plsc.load_gather(sidx_v, [eff])
            ih = plsc.load_gather(sidx_v, [hc])
            pick_l = (dl < dh) | ((dl == dh) & (il < ih))
            picked = jnp.where(pick_l, il, ih)
            plsc.store_scatter(otile_v, [lane * K + t], picked)
            l = jnp.where(pick_l, l - 1, l)
            h = jnp.where(pick_l, h, h + 1)
        pltpu.sync_copy(otile_v, out_hbm.at[pl.ds((b * N + base) * K, LN * K)])
        return carry

    lax.fori_loop(0, GRP, grp_body, 0)


@jax.jit
def kernel(x, Wa, ba, Wb, bb, Wc, bc):
    scores = pl.pallas_call(
        _scores_body,
        grid=(B,),
        in_specs=[
            pl.BlockSpec((1, N, L), lambda b: (b, 0, 0)),
            pl.BlockSpec((L, L), lambda b: (0, 0)),
            pl.BlockSpec((L,), lambda b: (0,)),
            pl.BlockSpec((L, L), lambda b: (0, 0)),
            pl.BlockSpec((L,), lambda b: (0,)),
            pl.BlockSpec((L, 1), lambda b: (0, 0)),
            pl.BlockSpec((1,), lambda b: (0,)),
        ],
        out_specs=pl.BlockSpec((1, N, 1), lambda b: (b, 0, 0)),
        out_shape=jax.ShapeDtypeStruct((B, N, 1), jnp.float32),
    )(x, Wa, ba, Wb, bb, Wc, bc)

    a_col = scores
    a_row = scores.reshape(B, 1, N)

    ranks, run_se = pl.pallas_call(
        _ranks_body,
        grid=(B, N // ROWS),
        in_specs=[
            pl.BlockSpec((1, ROWS, 1), lambda b, t: (b, t, 0)),
            pl.BlockSpec((1, 1, N), lambda b, t: (b, 0, 0)),
        ],
        out_specs=[
            pl.BlockSpec((1, ROWS, 1), lambda b, t: (b, t, 0)),
            pl.BlockSpec((1, ROWS, 1), lambda b, t: (b, t, 0)),
        ],
        out_shape=[
            jax.ShapeDtypeStruct((B, N, 1), jnp.int32),
            jax.ShapeDtypeStruct((B, N, 1), jnp.int32),
        ],
    )(a_col, a_row)

    sc_knn = pl.kernel(
        _sc_knn_body,
        out_type=jax.ShapeDtypeStruct((B * N * K,), jnp.int32),
        mesh=plsc.VectorSubcoreMesh(core_axis_name="c", subcore_axis_name="s"),
        compiler_params=pltpu.CompilerParams(needs_layout_passes=False),
        scratch_types=[
            pltpu.VMEM((N,), jnp.float32),
            pltpu.VMEM((N,), jnp.int32),
            pltpu.VMEM((N,), jnp.int32),
            pltpu.VMEM((N,), jnp.float32),
            pltpu.VMEM((N,), jnp.int32),
            pltpu.VMEM((N,), jnp.int32),
            pltpu.VMEM((LN * K,), jnp.int32),
        ],
    )
    nn_flat = sc_knn(scores.reshape(B * N), ranks.reshape(B * N),
                     run_se.reshape(B * N))

    nn_idx = nn_flat.reshape(B, N, K)
    center_idx = jnp.broadcast_to(
        jnp.arange(N, dtype=nn_idx.dtype)[None, :, None], (B, N, K))
    return jnp.stack((nn_idx, center_idx), axis=0)


# speedup vs baseline: 1.2454x; 1.0434x over previous
import jax
import jax.numpy as jnp
from jax import lax
from jax.experimental import pallas as pl
from jax.experimental.pallas import tpu as pltpu
from jax.experimental.pallas import tpu_sc as plsc

B, N, L, K = 4, 2048, 256, 16
ROWS = 512

NC, NS, LN = 2, 16, 16
NW = NC * NS
WPB = NW // B
RPW = N // WPB
GRP = RPW // LN


def _scores_body(x_ref, wa_ref, ba_ref, wb_ref, bb_ref, wc_ref, bc_ref, out_ref):
    xb = x_ref[0]
    a = jnp.tanh(jnp.dot(xb, wa_ref[...], preferred_element_type=jnp.float32)
                 + ba_ref[...][None, :])
    b = jax.nn.sigmoid(jnp.dot(xb, wb_ref[...], preferred_element_type=jnp.float32)
                       + bb_ref[...][None, :])
    logits = (jnp.dot(a * b, wc_ref[...], preferred_element_type=jnp.float32)
              + bc_ref[...][None, :])
    m = jnp.max(logits, axis=0, keepdims=True)
    e = jnp.exp(logits - m)
    s = jnp.sum(e, axis=0, keepdims=True)
    out_ref[...] = (e / s)[None]


def _ranks_body(acol_ref, arow_ref, rank_ref, rse_ref):
    ac = acol_ref[0]
    aa = arow_ref[0]
    t = pl.program_id(1)
    jj = lax.broadcasted_iota(jnp.int32, (ROWS, N), 1)
    ii = lax.broadcasted_iota(jnp.int32, (ROWS, N), 0) + t * ROWS
    eq = aa == ac
    lt = (aa < ac).astype(jnp.int32)
    eq_lt = (eq & (jj < ii)).astype(jnp.int32)
    lt_sum = jnp.sum(lt, axis=1)
    eq_sum = jnp.sum(eq.astype(jnp.int32), axis=1)
    eqlt_sum = jnp.sum(eq_lt, axis=1)
    rank_ref[0] = (lt_sum + eqlt_sum)[:, None]
    rse_ref[0] = (lt_sum * 4096 + (lt_sum + eq_sum - 1))[:, None]


def _sc_knn_body(vals_hbm, ranks_hbm, rse_hbm, out_hbm,
                 vals_v, ranks_v, rse_src_v,
                 sval_v, sidx_v, srse_v, otile_v):
    cid = lax.axis_index("c")
    sid = lax.axis_index("s")
    wid = sid * NC + cid
    b = lax.rem(wid, B)
    seg = lax.div(wid, B)

    pltpu.sync_copy(vals_hbm.at[pl.ds(b * N, N)], vals_v)
    pltpu.sync_copy(ranks_hbm.at[pl.ds(b * N, N)], ranks_v)
    pltpu.sync_copy(rse_hbm.at[pl.ds(b * N, N)], rse_src_v)

    def scat_body(i, carry):
        r = ranks_v[pl.ds(i * LN, LN)]
        idx = lax.iota(jnp.int32, LN) + i * LN
        plsc.store_scatter(sval_v, [r], vals_v[pl.ds(i * LN, LN)])
        plsc.store_scatter(sidx_v, [r], idx)
        plsc.store_scatter(srse_v, [r], rse_src_v[pl.ds(i * LN, LN)])
        return carry

    lax.fori_loop(0, N // LN, scat_body, 0)

    lane = lax.iota(jnp.int32, LN)
    inf = jnp.full((LN,), jnp.inf, jnp.float32)

    def grp_body(g, carry):
        base = seg * RPW + g * LN
        vi = vals_v[pl.ds(base, LN)]
        p = ranks_v[pl.ds(base, LN)]
        l = p
        h = p + 1
        for t in range(K):
            lvalid = l >= 0
            hvalid = h < N
            lc = jnp.maximum(l, 0)
            hc = jnp.minimum(h, N - 1)
            vl = plsc.load_gather(sval_v, [lc])
            vh = plsc.load_gather(sval_v, [hc])
            dl = jnp.where(lvalid, jnp.abs(vi - vl), inf)
            dh = jnp.where(hvalid, jnp.abs(vi - vh), inf)
            rse_l = plsc.load_gather(srse_v, [lc])
            rs_l = jnp.right_shift(rse_l, 12)
            re_l = jnp.bitwise_and(rse_l, 4095)
            eff = rs_l + (jnp.minimum(re_l, p) - lc)
            il = plsc.load_gather(sidx_v, [eff])
            ih = plsc.load_gather(sidx_v, [hc])
            pick_l = (dl < dh) | ((dl == dh) & (il < ih))
            picked = jnp.where(pick_l, il, ih)
            plsc.store_scatter(otile_v, [lane * K + t], picked)
            l = jnp.where(pick_l, l - 1, l)
            h = jnp.where(pick_l, h, h + 1)
        pltpu.sync_copy(otile_v, out_hbm.at[pl.ds((b * N + base) * K, LN * K)])
        return carry

    lax.fori_loop(0, GRP, grp_body, 0)


@jax.jit
def kernel(x, Wa, ba, Wb, bb, Wc, bc):
    scores = pl.pallas_call(
        _scores_body,
        grid=(B,),
        in_specs=[
            pl.BlockSpec((1, N, L), lambda b: (b, 0, 0)),
            pl.BlockSpec((L, L), lambda b: (0, 0)),
            pl.BlockSpec((L,), lambda b: (0,)),
            pl.BlockSpec((L, L), lambda b: (0, 0)),
            pl.BlockSpec((L,), lambda b: (0,)),
            pl.BlockSpec((L, 1), lambda b: (0, 0)),
            pl.BlockSpec((1,), lambda b: (0,)),
        ],
        out_specs=pl.BlockSpec((1, N, 1), lambda b: (b, 0, 0)),
        out_shape=jax.ShapeDtypeStruct((B, N, 1), jnp.float32),
    )(x, Wa, ba, Wb, bb, Wc, bc)

    a_col = scores
    a_row = scores.reshape(B, 1, N)

    ranks, run_se = pl.pallas_call(
        _ranks_body,
        grid=(B, N // ROWS),
        in_specs=[
            pl.BlockSpec((1, ROWS, 1), lambda b, t: (b, t, 0)),
            pl.BlockSpec((1, 1, N), lambda b, t: (b, 0, 0)),
        ],
        out_specs=[
            pl.BlockSpec((1, ROWS, 1), lambda b, t: (b, t, 0)),
            pl.BlockSpec((1, ROWS, 1), lambda b, t: (b, t, 0)),
        ],
        out_shape=[
            jax.ShapeDtypeStruct((B, N, 1), jnp.int32),
            jax.ShapeDtypeStruct((B, N, 1), jnp.int32),
        ],
    )(a_col, a_row)

    sc_knn = pl.kernel(
        _sc_knn_body,
        out_type=jax.ShapeDtypeStruct((B * N * K,), jnp.int32),
        mesh=plsc.VectorSubcoreMesh(core_axis_name="c", subcore_axis_name="s"),
        compiler_params=pltpu.CompilerParams(needs_layout_passes=False),
        scratch_types=[
            pltpu.VMEM((N,), jnp.float32),
            pltpu.VMEM((N,), jnp.int32),
            pltpu.VMEM((N,), jnp.int32),
            pltpu.VMEM((N,), jnp.float32),
            pltpu.VMEM((N,), jnp.int32),
            pltpu.VMEM((N,), jnp.int32),
            pltpu.VMEM((LN * K,), jnp.int32),
        ],
    )
    nn_flat = sc_knn(scores.reshape(B * N), ranks.reshape(B * N),
                     run_se.reshape(B * N))

    nn_idx = nn_flat.reshape(B, N, K)
    center_idx = jnp.broadcast_to(
        jnp.arange(N, dtype=nn_idx.dtype)[None, :, None], (B, N, K))
    return jnp.stack((nn_idx, center_idx), axis=0)


# speedup vs baseline: 1.2611x; 1.0126x over previous
import jax
import jax.numpy as jnp
from jax import lax
from jax.experimental import pallas as pl
from jax.experimental.pallas import tpu as pltpu
from jax.experimental.pallas import tpu_sc as plsc

B, N, L, K = 4, 2048, 256, 16
ROWS = 1024

NC, NS, LN = 2, 16, 16
NW = NC * NS
WPB = NW // B
RPW = N // WPB
GRP = RPW // LN


def _scores_body(x_ref, wa_ref, ba_ref, wb_ref, bb_ref, wc_ref, bc_ref, out_ref):
    xb = x_ref[0]
    a = jnp.tanh(jnp.dot(xb, wa_ref[...], preferred_element_type=jnp.float32)
                 + ba_ref[...][None, :])
    b = jax.nn.sigmoid(jnp.dot(xb, wb_ref[...], preferred_element_type=jnp.float32)
                       + bb_ref[...][None, :])
    logits = (jnp.dot(a * b, wc_ref[...], preferred_element_type=jnp.float32)
              + bc_ref[...][None, :])
    m = jnp.max(logits, axis=0, keepdims=True)
    e = jnp.exp(logits - m)
    s = jnp.sum(e, axis=0, keepdims=True)
    out_ref[...] = (e / s)[None]


def _ranks_body(acol_ref, arow_ref, rank_ref, rse_ref):
    ac = acol_ref[0]
    aa = arow_ref[0]
    t = pl.program_id(1)
    jj = lax.broadcasted_iota(jnp.int32, (ROWS, N), 1)
    ii = lax.broadcasted_iota(jnp.int32, (ROWS, N), 0) + t * ROWS
    eq = aa == ac
    lt = (aa < ac).astype(jnp.int32)
    eq_lt = (eq & (jj < ii)).astype(jnp.int32)
    lt_sum = jnp.sum(lt, axis=1)
    eq_sum = jnp.sum(eq.astype(jnp.int32), axis=1)
    eqlt_sum = jnp.sum(eq_lt, axis=1)
    rank_ref[0] = (lt_sum + eqlt_sum)[:, None]
    rse_ref[0] = (lt_sum * 4096 + (lt_sum + eq_sum - 1))[:, None]


def _sc_knn_body(vals_hbm, ranks_hbm, rse_hbm, out_hbm,
                 vals_v, ranks_v, rse_src_v,
                 sval_v, sidx_v, srse_v, otile_v):
    cid = lax.axis_index("c")
    sid = lax.axis_index("s")
    wid = sid * NC + cid
    b = lax.rem(wid, B)
    seg = lax.div(wid, B)

    pltpu.sync_copy(vals_hbm.at[pl.ds(b * N, N)], vals_v)
    pltpu.sync_copy(ranks_hbm.at[pl.ds(b * N, N)], ranks_v)
    pltpu.sync_copy(rse_hbm.at[pl.ds(b * N, N)], rse_src_v)

    def scat_body(i, carry):
        r = ranks_v[pl.ds(i * LN, LN)]
        idx = lax.iota(jnp.int32, LN) + i * LN
        plsc.store_scatter(sval_v, [r], vals_v[pl.ds(i * LN, LN)])
        plsc.store_scatter(sidx_v, [r], idx)
        plsc.store_scatter(srse_v, [r], rse_src_v[pl.ds(i * LN, LN)])
        return carry

    lax.fori_loop(0, N // LN, scat_body, 0)

    lane = lax.iota(jnp.int32, LN)
    inf = jnp.full((LN,), jnp.inf, jnp.float32)

    def grp_body(g, carry):
        base = seg * RPW + g * LN
        vi = vals_v[pl.ds(base, LN)]
        p = ranks_v[pl.ds(base, LN)]
        l = p
        h = p + 1
        for t in range(K):
            lvalid = l >= 0
            hvalid = h < N
            lc = jnp.maximum(l, 0)
            hc = jnp.minimum(h, N - 1)
            vl = plsc.load_gather(sval_v, [lc])
            vh = plsc.load_gather(sval_v, [hc])
            dl = jnp.where(lvalid, jnp.abs(vi - vl), inf)
            dh = jnp.where(hvalid, jnp.abs(vi - vh), inf)
            rse_l = plsc.load_gather(srse_v, [lc])
            rs_l = jnp.right_shift(rse_l, 12)
            re_l = jnp.bitwise_and(rse_l, 4095)
            eff = rs_l + (jnp.minimum(re_l, p) - lc)
            il = plsc.load_gather(sidx_v, [eff])
            ih = plsc.load_gather(sidx_v, [hc])
            pick_l = (dl < dh) | ((dl == dh) & (il < ih))
            picked = jnp.where(pick_l, il, ih)
            plsc.store_scatter(otile_v, [lane * K + t], picked)
            l = jnp.where(pick_l, l - 1, l)
            h = jnp.where(pick_l, h, h + 1)
        pltpu.sync_copy(otile_v, out_hbm.at[pl.ds((b * N + base) * K, LN * K)])
        return carry

    lax.fori_loop(0, GRP, grp_body, 0)


@jax.jit
def kernel(x, Wa, ba, Wb, bb, Wc, bc):
    scores = pl.pallas_call(
        _scores_body,
        grid=(B,),
        in_specs=[
            pl.BlockSpec((1, N, L), lambda b: (b, 0, 0)),
            pl.BlockSpec((L, L), lambda b: (0, 0)),
            pl.BlockSpec((L,), lambda b: (0,)),
            pl.BlockSpec((L, L), lambda b: (0, 0)),
            pl.BlockSpec((L,), lambda b: (0,)),
            pl.BlockSpec((L, 1), lambda b: (0, 0)),
            pl.BlockSpec((1,), lambda b: (0,)),
        ],
        out_specs=pl.BlockSpec((1, N, 1), lambda b: (b, 0, 0)),
        out_shape=jax.ShapeDtypeStruct((B, N, 1), jnp.float32),
    )(x, Wa, ba, Wb, bb, Wc, bc)

    a_col = scores
    a_row = scores.reshape(B, 1, N)

    ranks, run_se = pl.pallas_call(
        _ranks_body,
        grid=(B, N // ROWS),
        in_specs=[
            pl.BlockSpec((1, ROWS, 1), lambda b, t: (b, t, 0)),
            pl.BlockSpec((1, 1, N), lambda b, t: (b, 0, 0)),
        ],
        out_specs=[
            pl.BlockSpec((1, ROWS, 1), lambda b, t: (b, t, 0)),
            pl.BlockSpec((1, ROWS, 1), lambda b, t: (b, t, 0)),
        ],
        out_shape=[
            jax.ShapeDtypeStruct((B, N, 1), jnp.int32),
            jax.ShapeDtypeStruct((B, N, 1), jnp.int32),
        ],
    )(a_col, a_row)

    sc_knn = pl.kernel(
        _sc_knn_body,
        out_type=jax.ShapeDtypeStruct((B * N * K,), jnp.int32),
        mesh=plsc.VectorSubcoreMesh(core_axis_name="c", subcore_axis_name="s"),
        compiler_params=pltpu.CompilerParams(needs_layout_passes=False),
        scratch_types=[
            pltpu.VMEM((N,), jnp.float32),
            pltpu.VMEM((N,), jnp.int32),
            pltpu.VMEM((N,), jnp.int32),
            pltpu.VMEM((N,), jnp.float32),
            pltpu.VMEM((N,), jnp.int32),
            pltpu.VMEM((N,), jnp.int32),
            pltpu.VMEM((LN * K,), jnp.int32),
        ],
    )
    nn_flat = sc_knn(scores.reshape(B * N), ranks.reshape(B * N),
                     run_se.reshape(B * N))

    nn_idx = nn_flat.reshape(B, N, K)
    center_idx = jnp.broadcast_to(
        jnp.arange(N, dtype=nn_idx.dtype)[None, :, None], (B, N, K))
    return jnp.stack((nn_idx, center_idx), axis=0)
